# baseline (speedup 1.0000x reference)
import functools

import jax
import jax.numpy as jnp
from jax import lax
from jax.experimental import pallas as pl
from jax.experimental.pallas import tpu as pltpu
from jax.experimental.pallas import tpu_sc as plsc

S = 2048
D = 1024
E = 8
DFF = 2048
CAP = 384
SLOTS = E * CAP
KEEP_PAD = 2304
SRC_ROWS = SLOTS + KEEP_PAD
LANEPAD = 128

NC = 2
NS = 16
NW = NC * NS
CHUNK = SLOTS // NW
TOK_T = S // NS
TOK_W = S // NW

_f32 = jnp.float32
_i32 = jnp.int32


_RB = 256
_KEEP_BLK0 = SLOTS // _RB


def _router_body(x_ref, wg_ref, logits_ref, src_ref, eidx_ref, xs_ref,
                 carry_ref):
    i = pl.program_id(0)

    @pl.when(i == 0)
    def _init():
        carry_ref[...] = jnp.zeros((1, LANEPAD), _f32)

    x = x_ref[...]
    wg = wg_ref[...]
    logits = jnp.dot(x, wg, preferred_element_type=_f32)
    logits_ref[...] = logits
    lane = lax.broadcasted_iota(_i32, (_RB, LANEPAD), 1)
    ml = jnp.where(lane < E, logits, _f32(-1e30))
    mx = jnp.max(ml, axis=1, keepdims=True)
    ex = jnp.exp(ml - mx)
    ssum = jnp.sum(ex, axis=1, keepdims=True)
    probs = ex / ssum
    pm = jnp.max(probs, axis=1, keepdims=True)
    cand = jnp.where((probs == pm) & (lane < E), lane, _i32(LANEPAD - 1))
    idx = jnp.min(cand, axis=1, keepdims=True)
    onehot = (lane == idx).astype(jnp.bfloat16)
    r = lax.broadcasted_iota(_i32, (_RB, _RB), 0)
    c = lax.broadcasted_iota(_i32, (_RB, _RB), 1)
    ltm = (r >= c).astype(jnp.bfloat16)
    carry = carry_ref[...]
    prio = jnp.dot(ltm, onehot, preferred_element_type=_f32) + carry
    carry_ref[...] = carry + jnp.sum(onehot.astype(_f32), axis=0,
                                     keepdims=True)
    p = jnp.sum(jnp.where(lane == idx, prio, 0.0), axis=1,
                keepdims=True).astype(_i32)
    routed = p <= CAP
    tok = lax.broadcasted_iota(_i32, (_RB, 1), 0) + i * _RB
    slot = idx * CAP + jnp.minimum(p, CAP) - 1
    src_ref[...] = jnp.where(routed, slot, SLOTS + tok)
    eidx_ref[...] = jnp.where(routed, idx, 0)
    xs_ref[...] = pm * x


_router = pl.pallas_call(
    _router_body,
    grid=(S // _RB,),
    in_specs=[
        pl.BlockSpec((_RB, D), lambda i: (i, 0)),
        pl.BlockSpec((D, LANEPAD), lambda i: (0, 0)),
    ],
    scratch_shapes=[pltpu.VMEM((1, LANEPAD), _f32)],
    out_specs=[
        pl.BlockSpec((_RB, LANEPAD), lambda i: (i, 0)),
        pl.BlockSpec((_RB, 1), lambda i: (i, 0)),
        pl.BlockSpec((_RB, 1), lambda i: (i, 0)),
        pl.BlockSpec((_RB, D), lambda i: (_KEEP_BLK0 + i, 0)),
    ],
    out_shape=[
        jax.ShapeDtypeStruct((S, LANEPAD), _f32),
        jax.ShapeDtypeStruct((S, 1), _i32),
        jax.ShapeDtypeStruct((S, 1), _i32),
        jax.ShapeDtypeStruct((SRC_ROWS, D), _f32),
    ],
)


_mesh = plsc.VectorSubcoreMesh(core_axis_name="c", subcore_axis_name="s",
                               num_cores=NC, num_subcores=NS)


@functools.partial(
    pl.kernel,
    out_type=jax.ShapeDtypeStruct((SLOTS, D), _f32),
    mesh=_mesh,
    scratch_types=[
        pltpu.VMEM((TOK_T,), _i32),
        pltpu.VMEM((TOK_T,), _i32),
        pltpu.VMEM((CHUNK,), _i32),
        pltpu.VMEM((CHUNK, D), _f32),
        pltpu.VMEM_SHARED((SRC_ROWS,), _i32),
        pltpu.SemaphoreType.DMA,
    ],
    compiler_params=pltpu.CompilerParams(needs_layout_passes=False),
)
def _dispatch(src_hbm, xs_hbm, xbuf_hbm, sidx_v, tok_v, cidx_v, rows_v,
              st_sh, sem):
    sid = lax.axis_index("s")
    wid = sid * NC + lax.axis_index("c")
    cbase = wid * CHUNK
    for g in range(CHUNK // 16):
        cidx_v[pl.ds(g * 16, 16)] = SLOTS + ((lax.iota(_i32, 16)
                                              + (cbase + g * 16)) & (S - 1))
    pltpu.sync_copy(cidx_v, st_sh.at[pl.ds(cbase, CHUNK)])
    tb = sid * TOK_T
    pltpu.sync_copy(src_hbm.at[pl.ds(tb, TOK_T)], sidx_v)
    for g in range(TOK_T // 16):
        tok_v[pl.ds(g * 16, 16)] = lax.iota(_i32, 16) + (SLOTS + tb + g * 16)
    plsc.subcore_barrier()
    pltpu.sync_copy(tok_v, st_sh.at[sidx_v])
    plsc.subcore_barrier()
    pltpu.sync_copy(st_sh.at[pl.ds(cbase, CHUNK)], cidx_v)
    pltpu.async_copy(xs_hbm.at[cidx_v], rows_v, sem).wait()
    pltpu.sync_copy(rows_v, xbuf_hbm.at[pl.ds(cbase, CHUNK)])


_DSPLIT = 2
_DH = DFF // _DSPLIT


def _ffn_body(xb_ref, xsf_ref, w1_ref, w2_ref, o_ref):
    del xsf_ref
    j = pl.program_id(1)
    h = jnp.dot(xb_ref[...], w1_ref[0], preferred_element_type=_f32)
    h = jnp.maximum(h, 0.0)
    part = jnp.dot(h, w2_ref[0], preferred_element_type=_f32)

    @pl.when(j == 0)
    def _set():
        o_ref[...] = part

    @pl.when(j > 0)
    def _acc():
        o_ref[...] += part


_ffn = pl.pallas_call(
    _ffn_body,
    grid=(E, _DSPLIT),
    in_specs=[
        pl.BlockSpec((CAP, D), lambda i, j: (i, 0)),
        pl.BlockSpec(memory_space=pltpu.MemorySpace.HBM),
        pl.BlockSpec((1, D, _DH), lambda i, j: (i, 0, j)),
        pl.BlockSpec((1, _DH, D), lambda i, j: (i, j, 0)),
    ],
    out_specs=pl.BlockSpec((CAP, D), lambda i, j: (i, 0)),
    out_shape=jax.ShapeDtypeStruct((SRC_ROWS, D), _f32),
    input_output_aliases={1: 0},
)


@functools.partial(
    pl.kernel,
    out_type=jax.ShapeDtypeStruct((S, D), _f32),
    mesh=_mesh,
    scratch_types=[
        pltpu.VMEM((TOK_W,), _i32),
        pltpu.VMEM((TOK_W, D), _f32),
        pltpu.SemaphoreType.DMA,
    ],
    compiler_params=pltpu.CompilerParams(needs_layout_passes=False),
)
def _combine(src_hbm, big_hbm, out_hbm, idx_v, rows_v, sem):
    wid = lax.axis_index("s") * NC + lax.axis_index("c")
    tbase = wid * TOK_W
    pltpu.sync_copy(src_hbm.at[pl.ds(tbase, TOK_W)], idx_v)
    pltpu.async_copy(big_hbm.at[idx_v], rows_v, sem).wait()
    pltpu.sync_copy(rows_v, out_hbm.at[pl.ds(tbase, TOK_W)])


def kernel(norm_data, Wg, W1, W2):
    x = norm_data.reshape(S, D).astype(_f32)
    wgp = jnp.pad(Wg.astype(_f32), ((0, 0), (0, LANEPAD - E)))
    logits128, src_idx, eidx, xs = _router(x, wgp)
    router_logits = logits128[:, :E].reshape(1, S, E)
    src1 = src_idx.reshape(S)
    xbuf = _dispatch(src1, xs)
    big = _ffn(xbuf, xs, W1, W2)
    out = _combine(src1, big)
    return (out.reshape(1, S, D), router_logits, eidx.reshape(1, S))


# speedup vs baseline: 1.0371x; 1.0371x over previous
import functools

import jax
import jax.numpy as jnp
from jax import lax
from jax.experimental import pallas as pl
from jax.experimental.pallas import tpu as pltpu
from jax.experimental.pallas import tpu_sc as plsc

S = 2048
D = 1024
E = 8
DFF = 2048
CAP = 384
SLOTS = E * CAP
KEEP_PAD = 2304
SRC_ROWS = SLOTS + KEEP_PAD
LANEPAD = 128

NC = 2
NS = 16
NW = NC * NS
CHUNK = SLOTS // NW
TOK_T = S // NS
TOK_W = S // NW

_f32 = jnp.float32
_i32 = jnp.int32


_RB = 256
_KEEP_BLK0 = SLOTS // _RB


def _router_body(x_ref, wg_ref, logits_ref, src_ref, eidx_ref, xs_ref,
                 carry_ref):
    i = pl.program_id(0)

    @pl.when(i == 0)
    def _init():
        carry_ref[...] = jnp.zeros((1, LANEPAD), _f32)

    x = x_ref[...]
    wg = wg_ref[...]
    logits = jnp.dot(x, wg, preferred_element_type=_f32)
    logits_ref[...] = logits
    lane = lax.broadcasted_iota(_i32, (_RB, LANEPAD), 1)
    ml = jnp.where(lane < E, logits, _f32(-1e30))
    mx = jnp.max(ml, axis=1, keepdims=True)
    ex = jnp.exp(ml - mx)
    ssum = jnp.sum(ex, axis=1, keepdims=True)
    probs = ex / ssum
    pm = jnp.max(probs, axis=1, keepdims=True)
    cand = jnp.where((probs == pm) & (lane < E), lane, _i32(LANEPAD - 1))
    idx = jnp.min(cand, axis=1, keepdims=True)
    onehot = (lane == idx).astype(jnp.bfloat16)
    r = lax.broadcasted_iota(_i32, (_RB, _RB), 0)
    c = lax.broadcasted_iota(_i32, (_RB, _RB), 1)
    ltm = (r >= c).astype(jnp.bfloat16)
    carry = carry_ref[...]
    prio = jnp.dot(ltm, onehot, preferred_element_type=_f32) + carry
    carry_ref[...] = carry + jnp.sum(onehot.astype(_f32), axis=0,
                                     keepdims=True)
    p = jnp.sum(jnp.where(lane == idx, prio, 0.0), axis=1,
                keepdims=True).astype(_i32)
    routed = p <= CAP
    tok = lax.broadcasted_iota(_i32, (_RB, 1), 0) + i * _RB
    slot = idx * CAP + jnp.minimum(p, CAP) - 1
    src_ref[...] = jnp.where(routed, slot, SLOTS + tok)
    eidx_ref[...] = jnp.where(routed, idx, 0)
    xs_ref[...] = pm * x


_router = pl.pallas_call(
    _router_body,
    grid=(S // _RB,),
    in_specs=[
        pl.BlockSpec((_RB, D), lambda i: (i, 0)),
        pl.BlockSpec((D, LANEPAD), lambda i: (0, 0)),
    ],
    scratch_shapes=[pltpu.VMEM((1, LANEPAD), _f32)],
    out_specs=[
        pl.BlockSpec((_RB, LANEPAD), lambda i: (i, 0)),
        pl.BlockSpec((_RB, 1), lambda i: (i, 0)),
        pl.BlockSpec((_RB, 1), lambda i: (i, 0)),
        pl.BlockSpec((_RB, D), lambda i: (_KEEP_BLK0 + i, 0)),
    ],
    out_shape=[
        jax.ShapeDtypeStruct((S, LANEPAD), _f32),
        jax.ShapeDtypeStruct((S, 1), _i32),
        jax.ShapeDtypeStruct((S, 1), _i32),
        jax.ShapeDtypeStruct((SRC_ROWS, D), _f32),
    ],
)


_mesh = plsc.VectorSubcoreMesh(core_axis_name="c", subcore_axis_name="s",
                               num_cores=NC, num_subcores=NS)


@functools.partial(
    pl.kernel,
    out_type=jax.ShapeDtypeStruct((SLOTS, D), _f32),
    mesh=_mesh,
    scratch_types=[
        pltpu.VMEM((TOK_T,), _i32),
        pltpu.VMEM((TOK_T,), _i32),
        pltpu.VMEM((CHUNK,), _i32),
        pltpu.VMEM((CHUNK, D), _f32),
        pltpu.VMEM_SHARED((SRC_ROWS,), _i32),
        pltpu.SemaphoreType.DMA,
    ],
    compiler_params=pltpu.CompilerParams(needs_layout_passes=False),
)
def _dispatch(src_hbm, xs_hbm, xbuf_hbm, sidx_v, tok_v, cidx_v, rows_v,
              st_sh, sem):
    sid = lax.axis_index("s")
    wid = sid * NC + lax.axis_index("c")
    cbase = wid * CHUNK
    for g in range(CHUNK // 16):
        cidx_v[pl.ds(g * 16, 16)] = SLOTS + ((lax.iota(_i32, 16)
                                              + (cbase + g * 16)) & (S - 1))
    pltpu.sync_copy(cidx_v, st_sh.at[pl.ds(cbase, CHUNK)])
    tb = sid * TOK_T
    pltpu.sync_copy(src_hbm.at[pl.ds(tb, TOK_T)], sidx_v)
    for g in range(TOK_T // 16):
        tok_v[pl.ds(g * 16, 16)] = lax.iota(_i32, 16) + (SLOTS + tb + g * 16)
    plsc.subcore_barrier()
    pltpu.sync_copy(tok_v, st_sh.at[sidx_v])
    plsc.subcore_barrier()
    pltpu.sync_copy(st_sh.at[pl.ds(cbase, CHUNK)], cidx_v)
    pltpu.async_copy(xs_hbm.at[cidx_v], rows_v, sem).wait()
    pltpu.sync_copy(rows_v, xbuf_hbm.at[pl.ds(cbase, CHUNK)])


def _ffn_body(xb_ref, xsf_ref, w1_ref, w2_ref, o_ref):
    del xsf_ref
    h = jnp.dot(xb_ref[...], w1_ref[0], preferred_element_type=_f32)
    h = jnp.maximum(h, 0.0)
    o_ref[...] = jnp.dot(h, w2_ref[0], preferred_element_type=_f32)


_ffn = pl.pallas_call(
    _ffn_body,
    grid=(E,),
    in_specs=[
        pl.BlockSpec((CAP, D), lambda i: (i, 0)),
        pl.BlockSpec(memory_space=pltpu.MemorySpace.HBM),
        pl.BlockSpec((1, D, DFF), lambda i: (i, 0, 0)),
        pl.BlockSpec((1, DFF, D), lambda i: (i, 0, 0)),
    ],
    out_specs=pl.BlockSpec((CAP, D), lambda i: (i, 0)),
    out_shape=jax.ShapeDtypeStruct((SRC_ROWS, D), _f32),
    input_output_aliases={1: 0},
)


@functools.partial(
    pl.kernel,
    out_type=jax.ShapeDtypeStruct((S, D), _f32),
    mesh=_mesh,
    scratch_types=[
        pltpu.VMEM((TOK_W,), _i32),
        pltpu.VMEM((TOK_W, D), _f32),
        pltpu.SemaphoreType.DMA,
    ],
    compiler_params=pltpu.CompilerParams(needs_layout_passes=False),
)
def _combine(src_hbm, big_hbm, out_hbm, idx_v, rows_v, sem):
    wid = lax.axis_index("s") * NC + lax.axis_index("c")
    tbase = wid * TOK_W
    pltpu.sync_copy(src_hbm.at[pl.ds(tbase, TOK_W)], idx_v)
    pltpu.async_copy(big_hbm.at[idx_v], rows_v, sem).wait()
    pltpu.sync_copy(rows_v, out_hbm.at[pl.ds(tbase, TOK_W)])


def kernel(norm_data, Wg, W1, W2):
    x = norm_data.reshape(S, D).astype(_f32)
    wgp = jnp.pad(Wg.astype(_f32), ((0, 0), (0, LANEPAD - E)))
    logits128, src_idx, eidx, xs = _router(x, wgp)
    router_logits = logits128[:, :E].reshape(1, S, E)
    src1 = src_idx.reshape(S)
    xbuf = _dispatch(src1, xs)
    big = _ffn(xbuf, xs, W1, W2)
    out = _combine(src1, big)
    return (out.reshape(1, S, D), router_logits, eidx.reshape(1, S))
